# baseline (speedup 1.0000x reference)
import functools

import jax
import jax.numpy as jnp
from jax import lax
from jax.experimental import pallas as pl
from jax.experimental.pallas import tpu as pltpu
from jax.experimental.pallas import tpu_sc as plsc

_N = 16384
_M = 1000
_L = 16
_SEG = 5
_RPS = 200


def _sc_body(predT_hbm, tgt_hbm, idx_hbm, out_hbm, idx16_v, lab_v, buf_sh,
             col_v, out_v, sem):
    cid = lax.axis_index("c")
    sid = lax.axis_index("s")

    @pl.when((cid == 0) & (sid == 0))
    def _work():
        pltpu.sync_copy(idx_hbm, idx16_v)
        pltpu.async_copy(tgt_hbm.at[idx16_v], lab_v, sem).wait()
        ridx = idx16_v[...][0]
        cbase = pl.multiple_of((ridx // 128) * 128, 128)
        cl = ridx % 128

        copies = [
            pltpu.async_copy(
                predT_hbm.at[pl.ds(s * _RPS, _RPS), pl.ds(cbase, 128)],
                buf_sh.at[pl.ds(s * _RPS, _RPS)], sem)
            for s in range(_SEG)
        ]
        for c in copies:
            c.wait()
        pltpu.sync_copy(buf_sh.at[:, cl], col_v)

        labv = lab_v[...]
        lane = lax.iota(jnp.int32, _L)
        acc = jnp.zeros((_L,), jnp.float32)
        sel = jnp.zeros((_L,), jnp.float32)
        for j in range(_M // _L):
            chunk = col_v[pl.ds(j * _L, _L)]
            pos = lane + (j * _L)
            acc = acc + chunk
            sel = sel + jnp.where(pos == labv, chunk, 0.0)
        chunk = col_v[pl.ds(_M - _L, _L)]
        pos = lane + (_M - _L)
        new = lane >= 8
        acc = acc + jnp.where(new, chunk, 0.0)
        sel = sel + jnp.where(new & (pos == labv), chunk, 0.0)

        total = acc[0]
        elem = sel[0]
        for i in range(1, _L):
            total = total + acc[i]
            elem = elem + sel[i]
        loss = _M * elem - total
        out_v[...] = jnp.full((_L,), loss, jnp.float32)
        pltpu.sync_copy(out_v, out_hbm)


_MESH = plsc.VectorSubcoreMesh(core_axis_name="c", subcore_axis_name="s")

_sc_call = functools.partial(
    pl.kernel,
    mesh=_MESH,
    out_type=jax.ShapeDtypeStruct((_L,), jnp.float32),
    scratch_types=[
        pltpu.VMEM((_L,), jnp.int32),
        pltpu.VMEM((_L,), jnp.int32),
        pltpu.VMEM_SHARED((_M, 128), jnp.float32),
        pltpu.VMEM((_M,), jnp.float32),
        pltpu.VMEM((_L,), jnp.float32),
        pltpu.SemaphoreType.DMA,
    ],
)(_sc_body)


def kernel(pred, target, rand_idx):
    idx16 = jnp.full((_L,), jnp.asarray(rand_idx, jnp.int32))
    tgt = jnp.asarray(target, jnp.int32)
    out = _sc_call(pred.T, tgt, idx16)
    return out[0]


# speedup vs baseline: 8.2825x; 8.2825x over previous
import jax
import jax.numpy as jnp
from jax.experimental import pallas as pl
from jax.experimental.pallas import tpu as pltpu

_N = 16384
_M = 1000


def _loss_body(s_ref, predT_ref, tgt_ref, out_ref):
    ridx = s_ref[0]
    labrow = (ridx // 128) % 8
    labcol = ridx % 128
    ti = jax.lax.broadcasted_iota(jnp.int32, (8, 128), 0)
    tj = jax.lax.broadcasted_iota(jnp.int32, (8, 128), 1)
    label = jnp.sum(jnp.where((ti == labrow) & (tj == labcol), tgt_ref[...], 0))

    c = ridx % 128
    blk = predT_ref[...]
    pi = jax.lax.broadcasted_iota(jnp.int32, (_M, 128), 0)
    pj = jax.lax.broadcasted_iota(jnp.int32, (_M, 128), 1)
    colmask = pj == c
    total = jnp.sum(jnp.where(colmask, blk, 0.0))
    elem = jnp.sum(jnp.where(colmask & (pi == label), blk, 0.0))
    out_ref[0, 0] = _M * elem - total


def kernel(pred, target, rand_idx):
    ridx = jnp.asarray(rand_idx, jnp.int32).reshape((1,))
    predT = pred.T
    tgt2d = jnp.asarray(target, jnp.int32).reshape(128, 128)
    out = pl.pallas_call(
        _loss_body,
        grid_spec=pltpu.PrefetchScalarGridSpec(
            num_scalar_prefetch=1,
            grid=(1,),
            in_specs=[
                pl.BlockSpec((_M, 128), lambda i, s: (0, s[0] // 128)),
                pl.BlockSpec((8, 128), lambda i, s: (s[0] // 1024, (s[0] % 1024) // 128)),
            ],
            out_specs=pl.BlockSpec(memory_space=pltpu.SMEM),
        ),
        out_shape=jax.ShapeDtypeStruct((1, 1), jnp.float32),
    )(ridx, predT, tgt2d)
    return out.reshape(())
